# Initial kernel scaffold; baseline (speedup 1.0000x reference)
import jax
import jax.numpy as jnp
from jax.experimental import pallas as pl


def kernel(features, edge_index, weight, bias):
    raise NotImplementedError("write your pallas kernel here")


# speedup vs baseline: 4.0720x; 4.0720x over previous
import functools

import jax
import jax.numpy as jnp
from jax import lax
from jax.experimental import pallas as pl
from jax.experimental.pallas import tpu as pltpu
from jax.experimental.pallas import tpu_sc as plsc

NC = 2
NS = 16
CHUNK = 128


def _sc_scatter(src_p, dst_p, features, zeros, *, n_pad, n_chunks):
    E_pad = src_p.shape[0]
    D = features.shape[1]
    rpt = n_pad // NS
    edges_per_tile = n_chunks * CHUNK
    mesh = plsc.VectorSubcoreMesh(core_axis_name="c", subcore_axis_name="s")

    @functools.partial(
        pl.kernel,
        mesh=mesh,
        out_type=jax.ShapeDtypeStruct((NC, n_pad, D), jnp.float32),
        scratch_types=[
            pltpu.VMEM_SHARED((n_pad, D), jnp.float32),
            pltpu.VMEM((CHUNK,), jnp.int32),
            pltpu.VMEM((CHUNK,), jnp.int32),
            pltpu.VMEM((CHUNK, D), jnp.float32),
            pltpu.SemaphoreType.DMA,
        ],
    )
    def sc_kernel(src_hbm, dst_hbm, feat_hbm, zeros_hbm, part_hbm,
                  agg, src_v, dst_v, rows_v, sem):
        c = lax.axis_index("c")
        s = lax.axis_index("s")
        wid = s * NC + c

        row0 = pl.multiple_of(s * rpt, 8)
        pltpu.sync_copy(zeros_hbm.at[pl.ds(row0, rpt)], agg.at[pl.ds(row0, rpt)])
        plsc.subcore_barrier()

        base0 = wid * edges_per_tile

        def step(i, carry):
            base = pl.multiple_of(base0 + i * CHUNK, CHUNK)
            pltpu.sync_copy(src_hbm.at[pl.ds(base, CHUNK)], src_v)
            pltpu.sync_copy(dst_hbm.at[pl.ds(base, CHUNK)], dst_v)
            pltpu.async_copy(feat_hbm.at[src_v], rows_v, sem).wait()
            pltpu.sync_copy(rows_v, agg.at[dst_v], add=True)
            return carry

        lax.fori_loop(0, n_chunks, step, 0)
        plsc.subcore_barrier()

        pltpu.sync_copy(agg.at[pl.ds(row0, rpt)], part_hbm.at[c, pl.ds(row0, rpt)])

    return sc_kernel(src_p, dst_p, features, zeros)


def _tc_matmul_body(p0_ref, p1_ref, f_ref, w_ref, b_ref, o_ref):
    agg = p0_ref[...] + p1_ref[...] + f_ref[...]
    o_ref[...] = (
        jnp.dot(agg, w_ref[...], preferred_element_type=jnp.float32) + b_ref[...]
    )


def kernel(features, edge_index, weight, bias):
    N, D = features.shape
    E = edge_index.shape[1]

    n_chunks = -(-E // (NC * NS * CHUNK))
    E_pad = n_chunks * NC * NS * CHUNK
    n_pad = -(-(N + 1) // (NS * 8)) * NS * 8

    src = edge_index[0]
    dst = edge_index[1]
    pad = E_pad - E
    src_p = jnp.pad(src, (0, pad))
    dst_p = jnp.pad(dst, (0, pad), constant_values=N)
    zeros = jnp.zeros((n_pad, D), jnp.float32)

    parts = _sc_scatter(src_p, dst_p, features, zeros,
                        n_pad=n_pad, n_chunks=n_chunks)
    p0 = parts[0, :N, :]
    p1 = parts[1, :N, :]

    BM = 512
    grid = -(-N // BM)
    h = pl.pallas_call(
        _tc_matmul_body,
        grid=(grid,),
        in_specs=[
            pl.BlockSpec((BM, D), lambda i: (i, 0)),
            pl.BlockSpec((BM, D), lambda i: (i, 0)),
            pl.BlockSpec((BM, D), lambda i: (i, 0)),
            pl.BlockSpec((D, D), lambda i: (0, 0)),
            pl.BlockSpec((1, D), lambda i: (0, 0)),
        ],
        out_specs=pl.BlockSpec((BM, D), lambda i: (i, 0)),
        out_shape=jax.ShapeDtypeStruct((N, D), jnp.float32),
    )(p0, p1, features, weight, bias.reshape(1, D))
    return h
